# baseline (speedup 1.0000x reference)
import numpy as np
import jax
import jax.numpy as jnp
from jax.experimental import pallas as pl
from jax.experimental.pallas import tpu as pltpu

B = 1; S = 2048; DIM = 1024; NH = 12; G = 4; HPG = 3; DK = 64; DV = 64
L = 32; DST = 16; LSEL = 64; NSEL = 16; WIN = 512
NCMP = (S - L) // DST + 1
NSB = S // LSEL
HID = max(1, DK // 2)
NCP = 128
SCALE = 1.0 / float(np.sqrt(DK))
NEG = float(np.finfo(np.float32).min)
PROJ = NH * DK + 6 * G * DK

_CQ = 0
_CKS = NH * DK
_CVS = _CKS + G * DK
_CKW = _CVS + G * DV
_CVW = _CKW + G * DK
_CKC = _CVW + G * DV
_CVC = _CKC + G * DK
_ROPE_STARTS = (
    [_CQ + i * DK for i in range(NH)]
    + [_CKS + i * DK for i in range(G)]
    + [_CKW + i * DK for i in range(G)]
    + [_CKC + i * DK for i in range(G)]
)

_M = np.zeros((NCP, NSB), np.float32)
for _j in range(NCMP):
    _toks = np.arange(_j * DST, _j * DST + L)
    _blks = _toks // LSEL
    for _m in np.unique(_blks):
        _M[_j, _m] = float(np.mean(_blks == _m))

_A = np.zeros((NCP, S), np.float32)
for _j in range(NCMP):
    _A[_j, _j * DST:_j * DST + L] = 1.0 / L

_E = np.zeros((NSB, S), np.float32)
for _m in range(NSB):
    _E[_m, _m * LSEL:(_m + 1) * LSEL] = 1.0

_half = DK // 2

_f32 = jnp.float32


def _dot(a, b, prec=None):
    return jax.lax.dot_general(a, b, (((1,), (0,)), ((), ())),
                               preferred_element_type=_f32, precision=prec)


def _dot_t(a, b, prec=None):
    return jax.lax.dot_general(a, b, (((1,), (1,)), ((), ())),
                               preferred_element_type=_f32, precision=prec)


_TQ1 = 256


def _proj_body(x_ref, w_ref, cos_ref, sin_ref, o_ref):
    acc = _dot(x_ref[...], w_ref[...])
    c = cos_ref[...]
    s = sin_ref[...]
    pieces = []
    col = 0
    rope = set(_ROPE_STARTS)
    while col < PROJ:
        if col in rope:
            x1 = acc[:, col:col + _half]
            x2 = acc[:, col + _half:col + DK]
            pieces.append(x1 * c - x2 * s)
            pieces.append(x1 * s + x2 * c)
            col += DK
        else:
            nxt = min([r for r in rope if r > col] + [PROJ])
            pieces.append(acc[:, col:nxt])
            col = nxt
    o_ref[...] = jnp.concatenate(pieces, axis=1)


def _cmpkv_body(a_ref, k_ref, v_ref, kc_ref, vc_ref):
    a = a_ref[...]
    hi = jax.lax.Precision.HIGHEST
    for g in range(G):
        kc_ref[g] = _dot(a, k_ref[:, g * DK:(g + 1) * DK], hi)
        vc_ref[g] = _dot(a, v_ref[:, g * DV:(g + 1) * DV], hi)


_TQ = 256
_WW = WIN + _TQ


def _fused_body(q_ref, kc_ref, vc_ref, m_ref, ks_ref, vs_ref, kw_ref, vw_ref,
                e_ref, w1_ref, b1_ref, w2_ref, b2_ref, wo_ref, out_ref,
                osel_scr):
    i = pl.program_id(0)
    T3 = HPG * _TQ

    def qcat_g(g):
        return jnp.concatenate(
            [q_ref[:, (g * HPG + h) * DK:(g * HPG + h + 1) * DK]
             for h in range(HPG)], axis=0)

    qcats = [qcat_g(g) for g in range(G)]

    posn = (i * _TQ
            + (jax.lax.broadcasted_iota(jnp.int32, (T3, NCP), 0) & (_TQ - 1)))
    jj = jax.lax.broadcasted_iota(jnp.int32, (T3, NCP), 1)
    mc = (posn >= jj * DST + (L - 1)) & (jj < NCMP)
    mm = jax.lax.broadcasted_iota(jnp.int32, (_TQ, NSB), 1)
    posq = i * _TQ + jax.lax.broadcasted_iota(jnp.int32, (_TQ, NSB), 0)
    blk = posq // LSEL
    allowed = (mm * LSEL) <= posq
    force = (mm == 0) | (mm == blk)
    ocmp = [None] * NH
    bms = [None] * G
    for g in range(G):
        sc = _dot_t(qcats[g], kc_ref[g]) * SCALE
        sm = jnp.where(mc, sc, NEG)
        mx = jnp.max(sm, axis=-1, keepdims=True)
        e = jnp.where(mc, jnp.exp(sm - mx), 0.0)
        p = e / jnp.maximum(jnp.sum(e, axis=-1, keepdims=True), 1e-9)
        o3 = _dot(p, vc_ref[g])
        for h in range(HPG):
            ocmp[g * HPG + h] = o3[h * _TQ:(h + 1) * _TQ]
        p_grp = (p[0:_TQ] + p[_TQ:2 * _TQ] + p[2 * _TQ:3 * _TQ])
        p_slc = _dot(p_grp, m_ref[...])
        p_adj = jnp.where(force, p_slc + 1e6, p_slc)
        p_adj = jnp.where(allowed, p_adj, -1e9)
        rank = jnp.zeros((_TQ, NSB), _f32)
        for j in range(NSB):
            pj = p_adj[:, j:j + 1]
            beats = (pj > p_adj) | ((pj == p_adj) & (mm > j))
            rank = rank + beats.astype(_f32)
        bms[g] = ((rank < NSEL) & allowed).astype(_f32)

    def sel_all(width):
        qpos = i * _TQ + jax.lax.broadcasted_iota(jnp.int32, (_TQ, width), 0)
        kpos = jax.lax.broadcasted_iota(jnp.int32, (_TQ, width), 1)
        causal = qpos >= kpos
        for g in range(G):
            tok = _dot(bms[g], e_ref[0:NSB, 0:width]) > 0.5
            madd = jnp.where(tok & causal, 0.0, NEG)
            madd3 = jnp.concatenate([madd, madd, madd], axis=0)
            ks = ks_ref[0:width, g * DK:(g + 1) * DK]
            vs = vs_ref[0:width, g * DV:(g + 1) * DV]
            ss = _dot_t(qcats[g], ks) * SCALE + madd3
            mx = jnp.max(ss, axis=-1, keepdims=True)
            e = jnp.exp(ss - mx)
            p = e / jnp.maximum(jnp.sum(e, axis=-1, keepdims=True), 1e-9)
            o3 = _dot(p, vs)
            for h in range(HPG):
                n = g * HPG + h
                osel_scr[:, n * DV:(n + 1) * DV] = o3[h * _TQ:(h + 1) * _TQ]

    for k in range(4):
        @pl.when(i // 2 == k)
        def _(k=k):
            sel_all(512 * (k + 1))

    base = jnp.maximum(i - (_WW // _TQ - 1), 0) * _TQ
    qposw = (i * _TQ
             + (jax.lax.broadcasted_iota(jnp.int32, (T3, _WW), 0) & (_TQ - 1)))
    kposw = base + jax.lax.broadcasted_iota(jnp.int32, (T3, _WW), 1)
    maddw = jnp.where((qposw >= kposw) & ((qposw - kposw) < WIN), 0.0, NEG)
    owin = [None] * NH
    for g in range(G):
        kw = kw_ref[pl.ds(base, _WW), g * DK:(g + 1) * DK]
        vw = vw_ref[pl.ds(base, _WW), g * DV:(g + 1) * DV]
        sw = _dot_t(qcats[g], kw) * SCALE + maddw
        mxw = jnp.max(sw, axis=-1, keepdims=True)
        ew = jnp.exp(sw - mxw)
        pw = ew / jnp.maximum(jnp.sum(ew, axis=-1, keepdims=True), 1e-9)
        o3 = _dot(pw, vw)
        for h in range(HPG):
            owin[g * HPG + h] = o3[h * _TQ:(h + 1) * _TQ]

    w1 = w1_ref[...]
    b1 = b1_ref[0:1, :]
    w2 = w2_ref[...]
    b2 = b2_ref[0:1, :]
    pieces = []
    for g in range(G):
        base_c = g * HPG * DK
        qp = (q_ref[:, base_c:base_c + DK]
              + q_ref[:, base_c + DK:base_c + 2 * DK]
              + q_ref[:, base_c + 2 * DK:base_c + 3 * DK]) * (1.0 / HPG)
        h1 = _dot_t(qp, w1) + b1
        h1 = h1 * jax.nn.sigmoid(h1)
        gl = _dot_t(h1, w2) + b2
        g0 = gl[:, 0:1]
        g1 = gl[:, 1:2]
        g2 = gl[:, 2:3]
        mx = jnp.maximum(jnp.maximum(g0, g1), g2)
        e0 = jnp.exp(g0 - mx)
        e1 = jnp.exp(g1 - mx)
        e2 = jnp.exp(g2 - mx)
        den = e0 + e1 + e2
        p0 = e0 / den
        p1 = e1 / den
        p2 = e2 / den
        mn = jnp.minimum(jnp.minimum(g0, g1), g2)
        mid = g0 + g1 + g2 - mx - mn
        peaked = (mx - mid) > 50.0
        a0 = (g0 >= g1) & (g0 >= g2)
        a1 = jnp.logical_not(a0) & (g1 >= g2)
        a2 = jnp.logical_not(a0) & jnp.logical_not(a1)
        p0 = jnp.where(peaked, a0.astype(_f32), p0)
        p1 = jnp.where(peaked, a1.astype(_f32), p1)
        p2 = jnp.where(peaked, a2.astype(_f32), p2)
        for h in range(HPG):
            n = g * HPG + h
            pieces.append(p0 * ocmp[n]
                          + p1 * osel_scr[:, n * DV:(n + 1) * DV]
                          + p2 * owin[n])
    comb = jnp.concatenate(pieces, axis=1)
    out_ref[...] = _dot(comb, wo_ref[...])


def kernel(x, WQ, WKsel, WVsel, WKwin, WVwin, WKcmp, WVcmp, Wout,
           fc1W, fc1b, fc2W, fc2b):
    x2 = x.reshape(S, DIM)
    wall_t = jnp.concatenate(
        [WQ, WKsel, WVsel, WKwin, WVwin, WKcmp, WVcmp], axis=0).T
    freqs = 1.0 / (10000.0 ** (jnp.arange(_half, dtype=jnp.float32) / _half))
    ang = jnp.arange(S, dtype=jnp.float32)[:, None] * freqs[None, :]
    cos = jnp.cos(ang)
    sin = jnp.sin(ang)

    proj = pl.pallas_call(
        _proj_body,
        grid=(S // _TQ1,),
        in_specs=[
            pl.BlockSpec((_TQ1, DIM), lambda i: (i, 0)),
            pl.BlockSpec((DIM, PROJ), lambda i: (0, 0)),
            pl.BlockSpec((_TQ1, _half), lambda i: (i, 0)),
            pl.BlockSpec((_TQ1, _half), lambda i: (i, 0)),
        ],
        out_specs=pl.BlockSpec((_TQ1, PROJ), lambda i: (i, 0)),
        out_shape=jax.ShapeDtypeStruct((S, PROJ), _f32),
        compiler_params=pltpu.CompilerParams(
            dimension_semantics=("parallel",)),
    )(x2, wall_t, cos, sin)

    qr = proj[:, _CQ:_CQ + NH * DK]
    ksel = proj[:, _CKS:_CKS + G * DK]
    vsel = proj[:, _CVS:_CVS + G * DV]
    kwin = proj[:, _CKW:_CKW + G * DK]
    vwin = proj[:, _CVW:_CVW + G * DV]
    kcr = proj[:, _CKC:_CKC + G * DK]
    vcr = proj[:, _CVC:_CVC + G * DV]

    avg = jnp.asarray(_A)
    kc, vc = pl.pallas_call(
        _cmpkv_body,
        in_specs=[
            pl.BlockSpec((NCP, S), lambda: (0, 0)),
            pl.BlockSpec((S, G * DK), lambda: (0, 0)),
            pl.BlockSpec((S, G * DV), lambda: (0, 0)),
        ],
        out_specs=[
            pl.BlockSpec((G, NCP, DK), lambda: (0, 0, 0)),
            pl.BlockSpec((G, NCP, DV), lambda: (0, 0, 0)),
        ],
        out_shape=[
            jax.ShapeDtypeStruct((G, NCP, DK), _f32),
            jax.ShapeDtypeStruct((G, NCP, DV), _f32),
        ],
    )(avg, kcr, vcr)

    mmap = jnp.asarray(_M)
    expand = jnp.asarray(_E)
    b1p = jnp.broadcast_to(fc1b[None, :], (8, HID))
    w2p = jnp.zeros((8, HID), _f32).at[:3].set(fc2W)
    b2p = jnp.zeros((8, 8), _f32).at[:, :3].set(
        jnp.broadcast_to(fc2b[None, :], (8, 3)))
    wout_t = Wout.T

    out = pl.pallas_call(
        _fused_body,
        grid=(S // _TQ,),
        in_specs=[
            pl.BlockSpec((_TQ, NH * DK), lambda i: (i, 0)),
            pl.BlockSpec((G, NCP, DK), lambda i: (0, 0, 0)),
            pl.BlockSpec((G, NCP, DV), lambda i: (0, 0, 0)),
            pl.BlockSpec((NCP, NSB), lambda i: (0, 0)),
            pl.BlockSpec((S, G * DK), lambda i: (0, 0)),
            pl.BlockSpec((S, G * DV), lambda i: (0, 0)),
            pl.BlockSpec((S, G * DK), lambda i: (0, 0)),
            pl.BlockSpec((S, G * DV), lambda i: (0, 0)),
            pl.BlockSpec((NSB, S), lambda i: (0, 0)),
            pl.BlockSpec((HID, DK), lambda i: (0, 0)),
            pl.BlockSpec((8, HID), lambda i: (0, 0)),
            pl.BlockSpec((8, HID), lambda i: (0, 0)),
            pl.BlockSpec((8, 8), lambda i: (0, 0)),
            pl.BlockSpec((NH * DV, DIM), lambda i: (0, 0)),
        ],
        out_specs=pl.BlockSpec((_TQ, DIM), lambda i: (i, 0)),
        out_shape=jax.ShapeDtypeStruct((S, DIM), _f32),
        scratch_shapes=[pltpu.VMEM((_TQ, NH * DV), _f32)],
        compiler_params=pltpu.CompilerParams(
            dimension_semantics=("arbitrary",)),
    )(qr, kc, vc, mmap, ksel, vsel, kwin, vwin,
      expand, fc1W, b1p, w2p, b2p, wout_t)

    return out.reshape(B, S, DIM)


# speedup vs baseline: 1.6222x; 1.6222x over previous
import numpy as np
import jax
import jax.numpy as jnp
from jax.experimental import pallas as pl
from jax.experimental.pallas import tpu as pltpu

B = 1; S = 2048; DIM = 1024; NH = 12; G = 4; HPG = 3; DK = 64; DV = 64
L = 32; DST = 16; LSEL = 64; NSEL = 16; WIN = 512
NCMP = (S - L) // DST + 1
NSB = S // LSEL
HID = max(1, DK // 2)
NCP = 128
SCALE = 1.0 / float(np.sqrt(DK))
NEG = float(np.finfo(np.float32).min)
PROJ = NH * DK + 6 * G * DK

_CQ = 0
_CKS = NH * DK
_CVS = _CKS + G * DK
_CKW = _CVS + G * DV
_CVW = _CKW + G * DK
_CKC = _CVW + G * DV
_CVC = _CKC + G * DK
_ROPE_STARTS = (
    [_CQ + i * DK for i in range(NH)]
    + [_CKS + i * DK for i in range(G)]
    + [_CKW + i * DK for i in range(G)]
    + [_CKC + i * DK for i in range(G)]
)

_M = np.zeros((NCP, NSB), np.float32)
for _j in range(NCMP):
    _toks = np.arange(_j * DST, _j * DST + L)
    _blks = _toks // LSEL
    for _m in np.unique(_blks):
        _M[_j, _m] = float(np.mean(_blks == _m))

_A = np.zeros((NCP, S), np.float32)
for _j in range(NCMP):
    _A[_j, _j * DST:_j * DST + L] = 1.0 / L

_E = np.zeros((NSB, S), np.float32)
for _m in range(NSB):
    _E[_m, _m * LSEL:(_m + 1) * LSEL] = 1.0

_half = DK // 2

_f32 = jnp.float32


def _dot(a, b, prec=None):
    return jax.lax.dot_general(a, b, (((1,), (0,)), ((), ())),
                               preferred_element_type=_f32, precision=prec)


def _dot_t(a, b, prec=None):
    return jax.lax.dot_general(a, b, (((1,), (1,)), ((), ())),
                               preferred_element_type=_f32, precision=prec)


_TQ1 = 256


def _proj_body(x_ref, w_ref, cos_ref, sin_ref, o_ref):
    acc = _dot(x_ref[...], w_ref[...])
    c = cos_ref[...]
    s = sin_ref[...]
    pieces = []
    col = 0
    rope = set(_ROPE_STARTS)
    while col < PROJ:
        if col in rope:
            x1 = acc[:, col:col + _half]
            x2 = acc[:, col + _half:col + DK]
            pieces.append(x1 * c - x2 * s)
            pieces.append(x1 * s + x2 * c)
            col += DK
        else:
            nxt = min([r for r in rope if r > col] + [PROJ])
            pieces.append(acc[:, col:nxt])
            col = nxt
    o_ref[...] = jnp.concatenate(pieces, axis=1)


def _cmpkv_body(a_ref, k_ref, v_ref, kc_ref, vc_ref):
    a = a_ref[...]
    hi = jax.lax.Precision.HIGHEST
    for g in range(G):
        kc_ref[g] = _dot(a, k_ref[:, g * DK:(g + 1) * DK], hi)
        vc_ref[g] = _dot(a, v_ref[:, g * DV:(g + 1) * DV], hi)


_TQ2 = 256


def _cmp_attn_body(q_ref, kc_ref, vc_ref, m_ref, ocmp_ref, bm_ref):
    i = pl.program_id(0)
    T3 = HPG * _TQ2
    posn = (i * _TQ2
            + (jax.lax.broadcasted_iota(jnp.int32, (T3, NCP), 0) & (_TQ2 - 1)))
    jj = jax.lax.broadcasted_iota(jnp.int32, (T3, NCP), 1)
    mc = (posn >= jj * DST + (L - 1)) & (jj < NCMP)
    for g in range(G):
        kc = kc_ref[g]
        vc = vc_ref[g]
        qcat = jnp.concatenate(
            [q_ref[:, (g * HPG + h) * DK:(g * HPG + h + 1) * DK]
             for h in range(HPG)], axis=0)
        sc = _dot_t(qcat, kc) * SCALE
        sm = jnp.where(mc, sc, NEG)
        mx = jnp.max(sm, axis=-1, keepdims=True)
        e = jnp.where(mc, jnp.exp(sm - mx), 0.0)
        p = e / jnp.maximum(jnp.sum(e, axis=-1, keepdims=True), 1e-9)
        o3 = _dot(p, vc)
        for h in range(HPG):
            n = g * HPG + h
            ocmp_ref[:, n * DV:(n + 1) * DV] = o3[h * _TQ2:(h + 1) * _TQ2]
        p_grp = (p[0:_TQ2] + p[_TQ2:2 * _TQ2] + p[2 * _TQ2:3 * _TQ2])
        p_slc = _dot(p_grp, m_ref[...])
        mm = jax.lax.broadcasted_iota(jnp.int32, (_TQ2, NSB), 1)
        posq = i * _TQ2 + jax.lax.broadcasted_iota(jnp.int32, (_TQ2, NSB), 0)
        blk = posq // LSEL
        allowed = (mm * LSEL) <= posq
        force = (mm == 0) | (mm == blk)
        p_adj = jnp.where(force, p_slc + 1e6, p_slc)
        p_adj = jnp.where(allowed, p_adj, -1e9)
        rank = jnp.zeros((_TQ2, NSB), _f32)
        for j in range(NSB):
            pj = p_adj[:, j:j + 1]
            beats = (pj > p_adj) | ((pj == p_adj) & (mm > j))
            rank = rank + beats.astype(_f32)
        sel = (rank < NSEL) & allowed
        bm_ref[:, g * NSB:(g + 1) * NSB] = sel.astype(_f32)


_TQ3 = 128


_WW = WIN + _TQ3


def _selwin_body(q_ref, ks_ref, vs_ref, kw_ref, vw_ref, bm_ref, e_ref,
                 osel_ref, owin_ref):
    i = pl.program_id(0)

    def qcat_g(g):
        return jnp.concatenate(
            [q_ref[:, (g * HPG + h) * DK:(g * HPG + h + 1) * DK]
             for h in range(HPG)], axis=0)

    def sel_all(width):
        qpos = i * _TQ3 + jax.lax.broadcasted_iota(jnp.int32, (_TQ3, width), 0)
        kpos = jax.lax.broadcasted_iota(jnp.int32, (_TQ3, width), 1)
        causal = qpos >= kpos
        for g in range(G):
            bm = bm_ref[:, g * NSB:(g + 1) * NSB]
            tok = _dot(bm, e_ref[0:NSB, 0:width]) > 0.5
            madd = jnp.where(tok & causal, 0.0, NEG)
            madd3 = jnp.concatenate([madd, madd, madd], axis=0)
            ks = ks_ref[0:width, g * DK:(g + 1) * DK]
            vs = vs_ref[0:width, g * DV:(g + 1) * DV]
            ss = _dot_t(qcat_g(g), ks) * SCALE + madd3
            mx = jnp.max(ss, axis=-1, keepdims=True)
            e = jnp.exp(ss - mx)
            p = e / jnp.maximum(jnp.sum(e, axis=-1, keepdims=True), 1e-9)
            o3 = _dot(p, vs)
            for h in range(HPG):
                n = g * HPG + h
                osel_ref[:, n * DV:(n + 1) * DV] = o3[h * _TQ3:(h + 1) * _TQ3]

    for k in range(4):
        @pl.when(i // 4 == k)
        def _(k=k):
            sel_all(512 * (k + 1))

    base = jnp.maximum(i - (_WW // _TQ3 - 1), 0) * _TQ3
    T3 = HPG * _TQ3
    qpos = (i * _TQ3
            + (jax.lax.broadcasted_iota(jnp.int32, (T3, _WW), 0) & (_TQ3 - 1)))
    kpos = base + jax.lax.broadcasted_iota(jnp.int32, (T3, _WW), 1)
    winm = (qpos >= kpos) & ((qpos - kpos) < WIN)
    maddw = jnp.where(winm, 0.0, NEG)
    for g in range(G):
        kw = kw_ref[pl.ds(base, _WW), g * DK:(g + 1) * DK]
        vw = vw_ref[pl.ds(base, _WW), g * DV:(g + 1) * DV]
        sw = _dot_t(qcat_g(g), kw) * SCALE + maddw
        mxw = jnp.max(sw, axis=-1, keepdims=True)
        ew = jnp.exp(sw - mxw)
        pw = ew / jnp.maximum(jnp.sum(ew, axis=-1, keepdims=True), 1e-9)
        o3 = _dot(pw, vw)
        for h in range(HPG):
            n = g * HPG + h
            owin_ref[:, n * DV:(n + 1) * DV] = o3[h * _TQ3:(h + 1) * _TQ3]


_TQ4 = 256


def _gate_out_body(q_ref, oc_ref, os_ref, ow_ref, w1_ref, b1_ref, w2_ref,
                   b2_ref, wo_ref, out_ref):
    w1 = w1_ref[...]
    b1 = b1_ref[0:1, :]
    w2 = w2_ref[...]
    b2 = b2_ref[0:1, :]
    pieces = []
    for g in range(G):
        base = g * HPG * DK
        qp = (q_ref[:, base:base + DK]
              + q_ref[:, base + DK:base + 2 * DK]
              + q_ref[:, base + 2 * DK:base + 3 * DK]) * (1.0 / HPG)
        h1 = _dot_t(qp, w1) + b1
        h1 = h1 * jax.nn.sigmoid(h1)
        gl = _dot_t(h1, w2) + b2
        g0 = gl[:, 0:1]
        g1 = gl[:, 1:2]
        g2 = gl[:, 2:3]
        mx = jnp.maximum(jnp.maximum(g0, g1), g2)
        e0 = jnp.exp(g0 - mx)
        e1 = jnp.exp(g1 - mx)
        e2 = jnp.exp(g2 - mx)
        den = e0 + e1 + e2
        p0 = e0 / den
        p1 = e1 / den
        p2 = e2 / den
        mn = jnp.minimum(jnp.minimum(g0, g1), g2)
        mid = g0 + g1 + g2 - mx - mn
        peaked = (mx - mid) > 50.0
        a0 = (g0 >= g1) & (g0 >= g2)
        a1 = jnp.logical_not(a0) & (g1 >= g2)
        a2 = jnp.logical_not(a0) & jnp.logical_not(a1)
        p0 = jnp.where(peaked, a0.astype(_f32), p0)
        p1 = jnp.where(peaked, a1.astype(_f32), p1)
        p2 = jnp.where(peaked, a2.astype(_f32), p2)
        for h in range(HPG):
            n = g * HPG + h
            sl = slice(n * DV, (n + 1) * DV)
            pieces.append(p0 * oc_ref[:, sl] + p1 * os_ref[:, sl]
                          + p2 * ow_ref[:, sl])
    comb = jnp.concatenate(pieces, axis=1)
    out_ref[...] = _dot(comb, wo_ref[...])


def kernel(x, WQ, WKsel, WVsel, WKwin, WVwin, WKcmp, WVcmp, Wout,
           fc1W, fc1b, fc2W, fc2b):
    x2 = x.reshape(S, DIM)
    wall_t = jnp.concatenate(
        [WQ, WKsel, WVsel, WKwin, WVwin, WKcmp, WVcmp], axis=0).T
    freqs = 1.0 / (10000.0 ** (jnp.arange(_half, dtype=jnp.float32) / _half))
    ang = jnp.arange(S, dtype=jnp.float32)[:, None] * freqs[None, :]
    cos = jnp.cos(ang)
    sin = jnp.sin(ang)

    proj = pl.pallas_call(
        _proj_body,
        grid=(S // _TQ1,),
        in_specs=[
            pl.BlockSpec((_TQ1, DIM), lambda i: (i, 0)),
            pl.BlockSpec((DIM, PROJ), lambda i: (0, 0)),
            pl.BlockSpec((_TQ1, _half), lambda i: (i, 0)),
            pl.BlockSpec((_TQ1, _half), lambda i: (i, 0)),
        ],
        out_specs=pl.BlockSpec((_TQ1, PROJ), lambda i: (i, 0)),
        out_shape=jax.ShapeDtypeStruct((S, PROJ), _f32),
        compiler_params=pltpu.CompilerParams(
            dimension_semantics=("parallel",)),
    )(x2, wall_t, cos, sin)

    qr = proj[:, _CQ:_CQ + NH * DK]
    ksel = proj[:, _CKS:_CKS + G * DK]
    vsel = proj[:, _CVS:_CVS + G * DV]
    kwin = proj[:, _CKW:_CKW + G * DK]
    vwin = proj[:, _CVW:_CVW + G * DV]
    kcr = proj[:, _CKC:_CKC + G * DK]
    vcr = proj[:, _CVC:_CVC + G * DV]

    avg = jnp.asarray(_A)
    kc, vc = pl.pallas_call(
        _cmpkv_body,
        in_specs=[
            pl.BlockSpec((NCP, S), lambda: (0, 0)),
            pl.BlockSpec((S, G * DK), lambda: (0, 0)),
            pl.BlockSpec((S, G * DV), lambda: (0, 0)),
        ],
        out_specs=[
            pl.BlockSpec((G, NCP, DK), lambda: (0, 0, 0)),
            pl.BlockSpec((G, NCP, DV), lambda: (0, 0, 0)),
        ],
        out_shape=[
            jax.ShapeDtypeStruct((G, NCP, DK), _f32),
            jax.ShapeDtypeStruct((G, NCP, DV), _f32),
        ],
    )(avg, kcr, vcr)

    mmap = jnp.asarray(_M)
    o_cmp, bmf = pl.pallas_call(
        _cmp_attn_body,
        grid=(S // _TQ2,),
        in_specs=[
            pl.BlockSpec((_TQ2, NH * DK), lambda i: (i, 0)),
            pl.BlockSpec((G, NCP, DK), lambda i: (0, 0, 0)),
            pl.BlockSpec((G, NCP, DV), lambda i: (0, 0, 0)),
            pl.BlockSpec((NCP, NSB), lambda i: (0, 0)),
        ],
        out_specs=[
            pl.BlockSpec((_TQ2, NH * DV), lambda i: (i, 0)),
            pl.BlockSpec((_TQ2, G * NSB), lambda i: (i, 0)),
        ],
        out_shape=[
            jax.ShapeDtypeStruct((S, NH * DV), _f32),
            jax.ShapeDtypeStruct((S, G * NSB), _f32),
        ],
        compiler_params=pltpu.CompilerParams(
            dimension_semantics=("parallel",)),
    )(qr, kc, vc, mmap)

    expand = jnp.asarray(_E)
    o_sel, o_win = pl.pallas_call(
        _selwin_body,
        grid=(S // _TQ3,),
        in_specs=[
            pl.BlockSpec((_TQ3, NH * DK), lambda i: (i, 0)),
            pl.BlockSpec((S, G * DK), lambda i: (0, 0)),
            pl.BlockSpec((S, G * DV), lambda i: (0, 0)),
            pl.BlockSpec((S, G * DK), lambda i: (0, 0)),
            pl.BlockSpec((S, G * DV), lambda i: (0, 0)),
            pl.BlockSpec((_TQ3, G * NSB), lambda i: (i, 0)),
            pl.BlockSpec((NSB, S), lambda i: (0, 0)),
        ],
        out_specs=[
            pl.BlockSpec((_TQ3, NH * DV), lambda i: (i, 0)),
            pl.BlockSpec((_TQ3, NH * DV), lambda i: (i, 0)),
        ],
        out_shape=[
            jax.ShapeDtypeStruct((S, NH * DV), _f32),
            jax.ShapeDtypeStruct((S, NH * DV), _f32),
        ],
        compiler_params=pltpu.CompilerParams(
            dimension_semantics=("parallel",)),
    )(qr, ksel, vsel, kwin, vwin, bmf, expand)

    b1p = jnp.broadcast_to(fc1b[None, :], (8, HID))
    w2p = jnp.zeros((8, HID), _f32).at[:3].set(fc2W)
    b2p = jnp.zeros((8, 8), _f32).at[:, :3].set(
        jnp.broadcast_to(fc2b[None, :], (8, 3)))
    wout_t = Wout.T

    out = pl.pallas_call(
        _gate_out_body,
        grid=(S // _TQ4,),
        in_specs=[
            pl.BlockSpec((_TQ4, NH * DK), lambda i: (i, 0)),
            pl.BlockSpec((_TQ4, NH * DV), lambda i: (i, 0)),
            pl.BlockSpec((_TQ4, NH * DV), lambda i: (i, 0)),
            pl.BlockSpec((_TQ4, NH * DV), lambda i: (i, 0)),
            pl.BlockSpec((HID, DK), lambda i: (0, 0)),
            pl.BlockSpec((8, HID), lambda i: (0, 0)),
            pl.BlockSpec((8, HID), lambda i: (0, 0)),
            pl.BlockSpec((8, 8), lambda i: (0, 0)),
            pl.BlockSpec((NH * DV, DIM), lambda i: (0, 0)),
        ],
        out_specs=pl.BlockSpec((_TQ4, DIM), lambda i: (i, 0)),
        out_shape=jax.ShapeDtypeStruct((S, DIM), _f32),
        compiler_params=pltpu.CompilerParams(
            dimension_semantics=("parallel",)),
    )(qr, o_cmp, o_sel, o_win, fc1W, b1p, w2p, b2p, wout_t)

    return out.reshape(B, S, DIM)


# speedup vs baseline: 1.6898x; 1.0417x over previous
import numpy as np
import jax
import jax.numpy as jnp
from jax.experimental import pallas as pl
from jax.experimental.pallas import tpu as pltpu

B = 1; S = 2048; DIM = 1024; NH = 12; G = 4; HPG = 3; DK = 64; DV = 64
L = 32; DST = 16; LSEL = 64; NSEL = 16; WIN = 512
NCMP = (S - L) // DST + 1
NSB = S // LSEL
HID = max(1, DK // 2)
NCP = 128
SCALE = 1.0 / float(np.sqrt(DK))
NEG = float(np.finfo(np.float32).min)
PROJ = NH * DK + 6 * G * DK

_CQ = 0
_CKS = NH * DK
_CVS = _CKS + G * DK
_CKW = _CVS + G * DV
_CVW = _CKW + G * DK
_CKC = _CVW + G * DV
_CVC = _CKC + G * DK
_ROPE_STARTS = (
    [_CQ + i * DK for i in range(NH)]
    + [_CKS + i * DK for i in range(G)]
    + [_CKW + i * DK for i in range(G)]
    + [_CKC + i * DK for i in range(G)]
)

_M = np.zeros((NCP, NSB), np.float32)
for _j in range(NCMP):
    _toks = np.arange(_j * DST, _j * DST + L)
    _blks = _toks // LSEL
    for _m in np.unique(_blks):
        _M[_j, _m] = float(np.mean(_blks == _m))

_A = np.zeros((NCP, S), np.float32)
for _j in range(NCMP):
    _A[_j, _j * DST:_j * DST + L] = 1.0 / L

_E = np.zeros((NSB, S), np.float32)
for _m in range(NSB):
    _E[_m, _m * LSEL:(_m + 1) * LSEL] = 1.0

_half = DK // 2

_f32 = jnp.float32


def _dot(a, b, prec=None):
    return jax.lax.dot_general(a, b, (((1,), (0,)), ((), ())),
                               preferred_element_type=_f32, precision=prec)


def _dot_t(a, b, prec=None):
    return jax.lax.dot_general(a, b, (((1,), (1,)), ((), ())),
                               preferred_element_type=_f32, precision=prec)


_TQ1 = 256


def _proj_body(x_ref, w_ref, cos_ref, sin_ref, o_ref):
    acc = _dot(x_ref[...], w_ref[...])
    c = cos_ref[...]
    s = sin_ref[...]
    pieces = []
    col = 0
    rope = set(_ROPE_STARTS)
    while col < PROJ:
        if col in rope:
            x1 = acc[:, col:col + _half]
            x2 = acc[:, col + _half:col + DK]
            pieces.append(x1 * c - x2 * s)
            pieces.append(x1 * s + x2 * c)
            col += DK
        else:
            nxt = min([r for r in rope if r > col] + [PROJ])
            pieces.append(acc[:, col:nxt])
            col = nxt
    o_ref[...] = jnp.concatenate(pieces, axis=1)


def _cmpkv_body(a_ref, k_ref, v_ref, kc_ref, vc_ref):
    a = a_ref[...]
    hi = jax.lax.Precision.HIGHEST
    for g in range(G):
        kc_ref[g] = _dot(a, k_ref[:, g * DK:(g + 1) * DK], hi)
        vc_ref[g] = _dot(a, v_ref[:, g * DV:(g + 1) * DV], hi)


_TQ2 = 256


def _cmp_attn_body(q_ref, kc_ref, vc_ref, m_ref, ocmp_ref, bm_ref):
    i = pl.program_id(0)
    T3 = HPG * _TQ2
    posn = (i * _TQ2
            + (jax.lax.broadcasted_iota(jnp.int32, (T3, NCP), 0) & (_TQ2 - 1)))
    jj = jax.lax.broadcasted_iota(jnp.int32, (T3, NCP), 1)
    mc = (posn >= jj * DST + (L - 1)) & (jj < NCMP)
    p_adjs = []
    for g in range(G):
        kc = kc_ref[g]
        vc = vc_ref[g]
        qcat = jnp.concatenate(
            [q_ref[:, (g * HPG + h) * DK:(g * HPG + h + 1) * DK]
             for h in range(HPG)], axis=0)
        sc = _dot_t(qcat, kc) * SCALE
        sm = jnp.where(mc, sc, NEG)
        mx = jnp.max(sm, axis=-1, keepdims=True)
        e = jnp.where(mc, jnp.exp(sm - mx), 0.0)
        p = e / jnp.maximum(jnp.sum(e, axis=-1, keepdims=True), 1e-9)
        o3 = _dot(p, vc)
        for h in range(HPG):
            n = g * HPG + h
            ocmp_ref[:, n * DV:(n + 1) * DV] = o3[h * _TQ2:(h + 1) * _TQ2]
        p_grp = (p[0:_TQ2] + p[_TQ2:2 * _TQ2] + p[2 * _TQ2:3 * _TQ2])
        p_slc = _dot(p_grp, m_ref[...])
        mm = jax.lax.broadcasted_iota(jnp.int32, (_TQ2, NSB), 1)
        posq = i * _TQ2 + jax.lax.broadcasted_iota(jnp.int32, (_TQ2, NSB), 0)
        blk = posq // LSEL
        allowed = (mm * LSEL) <= posq
        force = (mm == 0) | (mm == blk)
        p_adj = jnp.where(force, p_slc + 1e6, p_slc)
        p_adjs.append(jnp.where(allowed, p_adj, -1e9))
    pall = jnp.concatenate(p_adjs, axis=0)
    mm4 = jax.lax.broadcasted_iota(jnp.int32, (G * _TQ2, NSB), 1)
    rank = jnp.zeros((G * _TQ2, NSB), _f32)
    for j in range(NSB):
        pj = pall[:, j:j + 1]
        beats = (pj > pall) | ((pj == pall) & (mm4 > j))
        rank = rank + beats.astype(_f32)
    for g in range(G):
        sel = (rank[g * _TQ2:(g + 1) * _TQ2] < NSEL) & allowed
        bm_ref[:, g * NSB:(g + 1) * NSB] = sel.astype(_f32)


_TQ3 = 128


_WW = WIN + _TQ3


def _selwin_body(q_ref, ks_ref, vs_ref, kw_ref, vw_ref, bm_ref, e_ref,
                 osel_ref, owin_ref):
    i = pl.program_id(0)

    def qcat_g(g):
        return jnp.concatenate(
            [q_ref[:, (g * HPG + h) * DK:(g * HPG + h + 1) * DK]
             for h in range(HPG)], axis=0)

    def sel_all(width):
        qpos = i * _TQ3 + jax.lax.broadcasted_iota(jnp.int32, (_TQ3, width), 0)
        kpos = jax.lax.broadcasted_iota(jnp.int32, (_TQ3, width), 1)
        causal = qpos >= kpos
        for g in range(G):
            bm = bm_ref[:, g * NSB:(g + 1) * NSB]
            tok = _dot(bm, e_ref[0:NSB, 0:width]) > 0.5
            madd = jnp.where(tok & causal, 0.0, NEG)
            madd3 = jnp.concatenate([madd, madd, madd], axis=0)
            ks = ks_ref[0:width, g * DK:(g + 1) * DK]
            vs = vs_ref[0:width, g * DV:(g + 1) * DV]
            ss = _dot_t(qcat_g(g), ks) * SCALE + madd3
            mx = jnp.max(ss, axis=-1, keepdims=True)
            e = jnp.exp(ss - mx)
            p = e / jnp.maximum(jnp.sum(e, axis=-1, keepdims=True), 1e-9)
            o3 = _dot(p, vs)
            for h in range(HPG):
                n = g * HPG + h
                osel_ref[:, n * DV:(n + 1) * DV] = o3[h * _TQ3:(h + 1) * _TQ3]

    for k in range(4):
        @pl.when(i // 4 == k)
        def _(k=k):
            sel_all(512 * (k + 1))

    base = jnp.maximum(i - (_WW // _TQ3 - 1), 0) * _TQ3
    T3 = HPG * _TQ3
    qpos = (i * _TQ3
            + (jax.lax.broadcasted_iota(jnp.int32, (T3, _WW), 0) & (_TQ3 - 1)))
    kpos = base + jax.lax.broadcasted_iota(jnp.int32, (T3, _WW), 1)
    winm = (qpos >= kpos) & ((qpos - kpos) < WIN)
    maddw = jnp.where(winm, 0.0, NEG)
    for g in range(G):
        kw = kw_ref[pl.ds(base, _WW), g * DK:(g + 1) * DK]
        vw = vw_ref[pl.ds(base, _WW), g * DV:(g + 1) * DV]
        sw = _dot_t(qcat_g(g), kw) * SCALE + maddw
        mxw = jnp.max(sw, axis=-1, keepdims=True)
        ew = jnp.exp(sw - mxw)
        pw = ew / jnp.maximum(jnp.sum(ew, axis=-1, keepdims=True), 1e-9)
        o3 = _dot(pw, vw)
        for h in range(HPG):
            n = g * HPG + h
            owin_ref[:, n * DV:(n + 1) * DV] = o3[h * _TQ3:(h + 1) * _TQ3]


_TQ4 = 256


def _gate_out_body(q_ref, oc_ref, os_ref, ow_ref, w1_ref, b1_ref, w2_ref,
                   b2_ref, wo_ref, out_ref):
    w1 = w1_ref[...]
    b1 = b1_ref[0:1, :]
    w2 = w2_ref[...]
    b2 = b2_ref[0:1, :]
    pieces = []
    for g in range(G):
        base = g * HPG * DK
        qp = (q_ref[:, base:base + DK]
              + q_ref[:, base + DK:base + 2 * DK]
              + q_ref[:, base + 2 * DK:base + 3 * DK]) * (1.0 / HPG)
        h1 = _dot_t(qp, w1) + b1
        h1 = h1 * jax.nn.sigmoid(h1)
        gl = _dot_t(h1, w2) + b2
        g0 = gl[:, 0:1]
        g1 = gl[:, 1:2]
        g2 = gl[:, 2:3]
        mx = jnp.maximum(jnp.maximum(g0, g1), g2)
        e0 = jnp.exp(g0 - mx)
        e1 = jnp.exp(g1 - mx)
        e2 = jnp.exp(g2 - mx)
        den = e0 + e1 + e2
        p0 = e0 / den
        p1 = e1 / den
        p2 = e2 / den
        mn = jnp.minimum(jnp.minimum(g0, g1), g2)
        mid = g0 + g1 + g2 - mx - mn
        peaked = (mx - mid) > 50.0
        a0 = (g0 >= g1) & (g0 >= g2)
        a1 = jnp.logical_not(a0) & (g1 >= g2)
        a2 = jnp.logical_not(a0) & jnp.logical_not(a1)
        p0 = jnp.where(peaked, a0.astype(_f32), p0)
        p1 = jnp.where(peaked, a1.astype(_f32), p1)
        p2 = jnp.where(peaked, a2.astype(_f32), p2)
        for h in range(HPG):
            n = g * HPG + h
            sl = slice(n * DV, (n + 1) * DV)
            pieces.append(p0 * oc_ref[:, sl] + p1 * os_ref[:, sl]
                          + p2 * ow_ref[:, sl])
    comb = jnp.concatenate(pieces, axis=1)
    out_ref[...] = _dot(comb, wo_ref[...])


def kernel(x, WQ, WKsel, WVsel, WKwin, WVwin, WKcmp, WVcmp, Wout,
           fc1W, fc1b, fc2W, fc2b):
    x2 = x.reshape(S, DIM)
    wall_t = jnp.concatenate(
        [WQ, WKsel, WVsel, WKwin, WVwin, WKcmp, WVcmp], axis=0).T
    freqs = 1.0 / (10000.0 ** (jnp.arange(_half, dtype=jnp.float32) / _half))
    ang = jnp.arange(S, dtype=jnp.float32)[:, None] * freqs[None, :]
    cos = jnp.cos(ang)
    sin = jnp.sin(ang)

    proj = pl.pallas_call(
        _proj_body,
        grid=(S // _TQ1,),
        in_specs=[
            pl.BlockSpec((_TQ1, DIM), lambda i: (i, 0)),
            pl.BlockSpec((DIM, PROJ), lambda i: (0, 0)),
            pl.BlockSpec((_TQ1, _half), lambda i: (i, 0)),
            pl.BlockSpec((_TQ1, _half), lambda i: (i, 0)),
        ],
        out_specs=pl.BlockSpec((_TQ1, PROJ), lambda i: (i, 0)),
        out_shape=jax.ShapeDtypeStruct((S, PROJ), _f32),
        compiler_params=pltpu.CompilerParams(
            dimension_semantics=("parallel",)),
    )(x2, wall_t, cos, sin)

    qr = proj[:, _CQ:_CQ + NH * DK]
    ksel = proj[:, _CKS:_CKS + G * DK]
    vsel = proj[:, _CVS:_CVS + G * DV]
    kwin = proj[:, _CKW:_CKW + G * DK]
    vwin = proj[:, _CVW:_CVW + G * DV]
    kcr = proj[:, _CKC:_CKC + G * DK]
    vcr = proj[:, _CVC:_CVC + G * DV]

    avg = jnp.asarray(_A)
    kc, vc = pl.pallas_call(
        _cmpkv_body,
        in_specs=[
            pl.BlockSpec((NCP, S), lambda: (0, 0)),
            pl.BlockSpec((S, G * DK), lambda: (0, 0)),
            pl.BlockSpec((S, G * DV), lambda: (0, 0)),
        ],
        out_specs=[
            pl.BlockSpec((G, NCP, DK), lambda: (0, 0, 0)),
            pl.BlockSpec((G, NCP, DV), lambda: (0, 0, 0)),
        ],
        out_shape=[
            jax.ShapeDtypeStruct((G, NCP, DK), _f32),
            jax.ShapeDtypeStruct((G, NCP, DV), _f32),
        ],
    )(avg, kcr, vcr)

    mmap = jnp.asarray(_M)
    o_cmp, bmf = pl.pallas_call(
        _cmp_attn_body,
        grid=(S // _TQ2,),
        in_specs=[
            pl.BlockSpec((_TQ2, NH * DK), lambda i: (i, 0)),
            pl.BlockSpec((G, NCP, DK), lambda i: (0, 0, 0)),
            pl.BlockSpec((G, NCP, DV), lambda i: (0, 0, 0)),
            pl.BlockSpec((NCP, NSB), lambda i: (0, 0)),
        ],
        out_specs=[
            pl.BlockSpec((_TQ2, NH * DV), lambda i: (i, 0)),
            pl.BlockSpec((_TQ2, G * NSB), lambda i: (i, 0)),
        ],
        out_shape=[
            jax.ShapeDtypeStruct((S, NH * DV), _f32),
            jax.ShapeDtypeStruct((S, G * NSB), _f32),
        ],
        compiler_params=pltpu.CompilerParams(
            dimension_semantics=("parallel",)),
    )(qr, kc, vc, mmap)

    expand = jnp.asarray(_E)
    o_sel, o_win = pl.pallas_call(
        _selwin_body,
        grid=(S // _TQ3,),
        in_specs=[
            pl.BlockSpec((_TQ3, NH * DK), lambda i: (i, 0)),
            pl.BlockSpec((S, G * DK), lambda i: (0, 0)),
            pl.BlockSpec((S, G * DV), lambda i: (0, 0)),
            pl.BlockSpec((S, G * DK), lambda i: (0, 0)),
            pl.BlockSpec((S, G * DV), lambda i: (0, 0)),
            pl.BlockSpec((_TQ3, G * NSB), lambda i: (i, 0)),
            pl.BlockSpec((NSB, S), lambda i: (0, 0)),
        ],
        out_specs=[
            pl.BlockSpec((_TQ3, NH * DV), lambda i: (i, 0)),
            pl.BlockSpec((_TQ3, NH * DV), lambda i: (i, 0)),
        ],
        out_shape=[
            jax.ShapeDtypeStruct((S, NH * DV), _f32),
            jax.ShapeDtypeStruct((S, NH * DV), _f32),
        ],
        compiler_params=pltpu.CompilerParams(
            dimension_semantics=("parallel",)),
    )(qr, ksel, vsel, kwin, vwin, bmf, expand)

    b1p = jnp.broadcast_to(fc1b[None, :], (8, HID))
    w2p = jnp.zeros((8, HID), _f32).at[:3].set(fc2W)
    b2p = jnp.zeros((8, 8), _f32).at[:, :3].set(
        jnp.broadcast_to(fc2b[None, :], (8, 3)))
    wout_t = Wout.T

    out = pl.pallas_call(
        _gate_out_body,
        grid=(S // _TQ4,),
        in_specs=[
            pl.BlockSpec((_TQ4, NH * DK), lambda i: (i, 0)),
            pl.BlockSpec((_TQ4, NH * DV), lambda i: (i, 0)),
            pl.BlockSpec((_TQ4, NH * DV), lambda i: (i, 0)),
            pl.BlockSpec((_TQ4, NH * DV), lambda i: (i, 0)),
            pl.BlockSpec((HID, DK), lambda i: (0, 0)),
            pl.BlockSpec((8, HID), lambda i: (0, 0)),
            pl.BlockSpec((8, HID), lambda i: (0, 0)),
            pl.BlockSpec((8, 8), lambda i: (0, 0)),
            pl.BlockSpec((NH * DV, DIM), lambda i: (0, 0)),
        ],
        out_specs=pl.BlockSpec((_TQ4, DIM), lambda i: (i, 0)),
        out_shape=jax.ShapeDtypeStruct((S, DIM), _f32),
        compiler_params=pltpu.CompilerParams(
            dimension_semantics=("parallel",)),
    )(qr, o_cmp, o_sel, o_win, fc1W, b1p, w2p, b2p, wout_t)

    return out.reshape(B, S, DIM)
